# baseline (speedup 1.0000x reference)
import jax
import jax.numpy as jnp
from jax.experimental import pallas as pl
from jax.experimental.pallas import tpu as pltpu

_TEMP_INV = 20.0
_B = 1024
_D = 64
_N = 100000
_BN = 2048
_NBLK = (_N + _BN - 1) // _BN


def _fused_kernel(x_ref, t_ref, f_ref, out_ref, loss_ref,
                  xs_ref, se_ref, pk_ref):
    j = pl.program_id(0)

    @pl.when(j == 0)
    def _init():
        x = x_ref[...]
        nrm = jnp.sqrt(jnp.sum(x * x, axis=1, keepdims=True))
        xs_ref[...] = x / jnp.maximum(nrm, 1e-12)
        se_ref[...] = jnp.zeros_like(se_ref)
        pk_ref[...] = jnp.zeros_like(pk_ref)

    xn = xs_ref[...]
    f = f_ref[...]
    v = jax.lax.dot_general(
        xn, f, (((1,), (1,)), ((), ())),
        preferred_element_type=jnp.float32) * _TEMP_INV
    out_ref[...] = v

    col = j * _BN + jax.lax.broadcasted_iota(jnp.int32, (_B, _BN), 1)
    valid = col < _N
    e = jnp.where(valid, jnp.exp(v - _TEMP_INV), 0.0)
    se_ref[...] += jnp.sum(e, axis=1, keepdims=True)
    tmask = col == t_ref[...]
    pk_ref[...] += jnp.sum(jnp.where(tmask, v, 0.0), axis=1, keepdims=True)

    @pl.when(j == _NBLK - 1)
    def _fin():
        lse = _TEMP_INV + jnp.log(se_ref[...])
        loss_ref[0, 0] = jnp.sum(lse - pk_ref[...]) / _B


def kernel(inputs, targets, features):
    targets2d = targets.astype(jnp.int32).reshape(_B, 1)
    outputs, loss2d = pl.pallas_call(
        _fused_kernel,
        grid=(_NBLK,),
        in_specs=[
            pl.BlockSpec((_B, _D), lambda j: (0, 0)),
            pl.BlockSpec((_B, 1), lambda j: (0, 0)),
            pl.BlockSpec((_BN, _D), lambda j: (j, 0)),
        ],
        out_specs=[
            pl.BlockSpec((_B, _BN), lambda j: (0, j)),
            pl.BlockSpec(memory_space=pltpu.SMEM),
        ],
        out_shape=[
            jax.ShapeDtypeStruct((_B, _N), jnp.float32),
            jax.ShapeDtypeStruct((1, 1), jnp.float32),
        ],
        scratch_shapes=[
            pltpu.VMEM((_B, _D), jnp.float32),
            pltpu.VMEM((_B, 1), jnp.float32),
            pltpu.VMEM((_B, 1), jnp.float32),
        ],
    )(inputs, targets2d, features)
    loss = loss2d[0, 0]
    loss = jnp.where(jnp.isnan(loss), jnp.float32(0.0), loss)
    return (loss, outputs)
